# baseline (speedup 1.0000x reference)
import jax
import jax.numpy as jnp
from jax.experimental import pallas as pl
from jax.experimental.pallas import tpu as pltpu

_INDEX = 2
_ROWS, _COLS = 4096, 2048
_CHUNK = 128
_NCHUNKS = _ROWS // _CHUNK


def _body(x_hbm, o_hbm, buf, in_sems, out_sems):
    def in_copy(i):
        return pltpu.make_async_copy(
            x_hbm.at[_INDEX, pl.ds(i * _CHUNK, _CHUNK)],
            buf.at[pl.ds(i * _CHUNK, _CHUNK)],
            in_sems.at[i],
        )

    def out_copy(i):
        return pltpu.make_async_copy(
            buf.at[pl.ds(i * _CHUNK, _CHUNK)],
            o_hbm.at[pl.ds(i * _CHUNK, _CHUNK)],
            out_sems.at[i],
        )

    for i in range(_NCHUNKS):
        in_copy(i).start()
    for i in range(_NCHUNKS):
        in_copy(i).wait()
        out_copy(i).start()
    for i in range(_NCHUNKS):
        out_copy(i).wait()


def kernel(x):
    return pl.pallas_call(
        _body,
        in_specs=[pl.BlockSpec(memory_space=pl.ANY)],
        out_specs=pl.BlockSpec(memory_space=pl.ANY),
        out_shape=jax.ShapeDtypeStruct((_ROWS, _COLS), jnp.float32),
        scratch_shapes=[
            pltpu.VMEM((_ROWS, _COLS), jnp.float32),
            pltpu.SemaphoreType.DMA((_NCHUNKS,)),
            pltpu.SemaphoreType.DMA((_NCHUNKS,)),
        ],
    )(x)
